# baseline (speedup 1.0000x reference)
import functools

import jax
import jax.numpy as jnp
from jax import lax
from jax.experimental import pallas as pl
from jax.experimental.pallas import tpu as pltpu
from jax.experimental.pallas import tpu_sc as plsc

N = 10000
E = 320000
D = 128
NC = 2
NS = 16
NW = NC * NS
CH = 128
EP = 327680
CPW = EP // CH // NW
EP2 = EP + NW * CH
ACC_ROWS = 10112
RPT = 624
RPT_ACC = ACC_ROWS // NS
BR = 1000


def _sc_mesh():
    return plsc.VectorSubcoreMesh(core_axis_name="c", subcore_axis_name="s")


def _sc_propagate(hp, row_p, col_p):
    @functools.partial(
        pl.kernel,
        out_type=jax.ShapeDtypeStruct((NC, N, D), jnp.float32),
        mesh=_sc_mesh(),
        scratch_types=[
            pltpu.VMEM_SHARED((ACC_ROWS, D), jnp.float32),
            pltpu.VMEM((1, CH), jnp.int32),
            pltpu.VMEM((1, CH), jnp.int32),
            pltpu.VMEM((1, CH), jnp.int32),
            pltpu.VMEM((1, CH), jnp.int32),
            pltpu.VMEM((CH, D), jnp.float32),
            pltpu.VMEM((CH, D), jnp.float32),
            pltpu.SemaphoreType.DMA,
            pltpu.SemaphoreType.DMA,
        ],
    )
    def prop_kernel(h_hbm, row_hbm, col_hbm, out_hbm,
                    acc, row_v0, row_v1, col_v0, col_v1,
                    rows_v0, rows_v1, sem0, sem1):
        row_vs = (row_v0, row_v1)
        col_vs = (col_v0, col_v1)
        rows_vs = (rows_v0, rows_v1)
        sems = (sem0, sem1)
        cid = lax.axis_index("c")
        sid = lax.axis_index("s")
        wid = sid * NC + cid
        base_r = sid * RPT
        pltpu.sync_copy(h_hbm.at[pl.ds(base_r, RPT)],
                        acc.at[pl.ds(base_r, RPT)])

        @pl.when(sid == NS - 1)
        def _():
            pltpu.sync_copy(h_hbm.at[pl.ds(NS * RPT, N - NS * RPT)],
                            acc.at[pl.ds(NS * RPT, N - NS * RPT)])

        plsc.subcore_barrier()

        def load_idx(g, b):
            base = (g * NW + wid) * CH
            pltpu.sync_copy(row_hbm.at[pl.ds(base, CH)], row_vs[b].at[0])
            pltpu.sync_copy(col_hbm.at[pl.ds(base, CH)], col_vs[b].at[0])

        def start_gather(b):
            pltpu.async_copy(h_hbm.at[row_vs[b].at[0]], rows_vs[b], sems[b])

        def wait_gather(b):
            pltpu.make_async_copy(h_hbm.at[row_vs[b].at[0]], rows_vs[b],
                                  sems[b]).wait()

        def scatter(b):
            pltpu.sync_copy(rows_vs[b], acc.at[col_vs[b].at[0]], add=True)

        load_idx(0, 0)
        start_gather(0)

        def body(gg, carry):
            g0 = gg * 2
            load_idx(g0 + 1, 1)
            start_gather(1)
            wait_gather(0)
            scatter(0)
            load_idx(g0 + 2, 0)
            start_gather(0)
            wait_gather(1)
            scatter(1)
            return carry

        lax.fori_loop(0, CPW // 2, body, 0)
        wait_gather(0)
        plsc.subcore_barrier()
        pltpu.sync_copy(acc.at[pl.ds(base_r, RPT)],
                        out_hbm.at[cid, pl.ds(base_r, RPT)])

        @pl.when(sid == NS - 1)
        def _():
            pltpu.sync_copy(acc.at[pl.ds(NS * RPT, N - NS * RPT)],
                            out_hbm.at[cid, pl.ds(NS * RPT, N - NS * RPT)])

    return prop_kernel(hp, row_p, col_p)


def _tc_stage1(x, W1, degp):
    def body(x_ref, w_ref, d_ref, h_ref, dis_ref):
        d0 = d_ref[0, :, 0:1]
        d1 = d_ref[1, :, 0:1]
        dis = lax.rsqrt(d0 + d1 - 1.0)
        h = lax.dot_general(x_ref[...], w_ref[...],
                            (((1,), (1,)), ((), ())),
                            preferred_element_type=jnp.float32)
        h_ref[...] = h * dis
        dis_ref[...] = jnp.broadcast_to(dis, (BR, D))

    return pl.pallas_call(
        body,
        grid=(N // BR,),
        in_specs=[
            pl.BlockSpec((BR, D), lambda i: (i, 0)),
            pl.BlockSpec((D, D), lambda i: (0, 0)),
            pl.BlockSpec((NC, BR, D), lambda i: (0, i, 0)),
        ],
        out_specs=[
            pl.BlockSpec((BR, D), lambda i: (i, 0)),
            pl.BlockSpec((BR, D), lambda i: (i, 0)),
        ],
        out_shape=[
            jax.ShapeDtypeStruct((N, D), jnp.float32),
            jax.ShapeDtypeStruct((N, D), jnp.float32),
        ],
    )(x, W1, degp)


def _tc_stage2(p, hp, dis, b1, W2):
    def body(p_ref, hp_ref, dis_ref, b_ref, w_ref, o_ref):
        s = p_ref[0] + p_ref[1] - hp_ref[...]
        h = s * dis_ref[...] + b_ref[...]
        h = jnp.maximum(h, 0.0)
        o = lax.dot_general(h, w_ref[...],
                            (((1,), (1,)), ((), ())),
                            preferred_element_type=jnp.float32)
        o_ref[...] = o * dis_ref[...]

    return pl.pallas_call(
        body,
        grid=(N // BR,),
        in_specs=[
            pl.BlockSpec((NC, BR, D), lambda i: (0, i, 0)),
            pl.BlockSpec((BR, D), lambda i: (i, 0)),
            pl.BlockSpec((BR, D), lambda i: (i, 0)),
            pl.BlockSpec((1, D), lambda i: (0, 0)),
            pl.BlockSpec((D, D), lambda i: (0, 0)),
        ],
        out_specs=pl.BlockSpec((BR, D), lambda i: (i, 0)),
        out_shape=jax.ShapeDtypeStruct((N, D), jnp.float32),
    )(p, hp, dis, b1, W2)


def _tc_stage3(q, hp, dis, b2):
    def body(q_ref, hp_ref, dis_ref, b_ref, o_ref):
        s = q_ref[0] + q_ref[1] - hp_ref[...]
        o_ref[...] = s * dis_ref[...] + b_ref[...]

    return pl.pallas_call(
        body,
        grid=(N // BR,),
        in_specs=[
            pl.BlockSpec((NC, BR, D), lambda i: (0, i, 0)),
            pl.BlockSpec((BR, D), lambda i: (i, 0)),
            pl.BlockSpec((BR, D), lambda i: (i, 0)),
            pl.BlockSpec((1, D), lambda i: (0, 0)),
        ],
        out_specs=pl.BlockSpec((BR, D), lambda i: (i, 0)),
        out_shape=jax.ShapeDtypeStruct((N, D), jnp.float32),
    )(q, hp, dis, b2)


def kernel(x, edge_index, W1, b1, W2, b2):
    row = edge_index[0].astype(jnp.int32)
    col = edge_index[1].astype(jnp.int32)
    pad = EP2 - E
    row_p = jnp.concatenate([row, jnp.zeros((pad,), jnp.int32)])
    col_p = jnp.concatenate([col, jnp.full((pad,), N, jnp.int32)])

    ones = jnp.ones((N, D), jnp.float32)
    degp = _sc_propagate(ones, row_p, col_p)
    h1p, dis = _tc_stage1(x, W1, degp)
    p = _sc_propagate(h1p, row_p, col_p)
    h2p = _tc_stage2(p, h1p, dis, b1.reshape(1, D), W2)
    q = _sc_propagate(h2p, row_p, col_p)
    return _tc_stage3(q, h2p, dis, b2.reshape(1, D))


# speedup vs baseline: 1.5871x; 1.5871x over previous
import functools

import jax
import jax.numpy as jnp
from jax import lax
from jax.experimental import pallas as pl
from jax.experimental.pallas import tpu as pltpu
from jax.experimental.pallas import tpu_sc as plsc

N = 10000
E = 320000
D = 128
NC = 2
NS = 16
NW = NC * NS
CH = 128
EP = 327680
CPW = EP // CH // NW
EP2 = EP + NW * CH
ACC_ROWS = 10112
RPT = 624
RPT_ACC = ACC_ROWS // NS
BR = 1024
GR = (N + BR - 1) // BR


def _sc_mesh():
    return plsc.VectorSubcoreMesh(core_axis_name="c", subcore_axis_name="s")


HN = 10240


def _sc_degree(col_p, zeros_hn):
    @functools.partial(
        pl.kernel,
        out_type=jax.ShapeDtypeStruct((NW, HN), jnp.float32),
        mesh=_sc_mesh(),
        scratch_types=[
            pltpu.VMEM((HN,), jnp.float32),
            pltpu.VMEM((1, CH), jnp.int32),
        ],
        compiler_params=pltpu.CompilerParams(needs_layout_passes=False),
    )
    def deg_kernel(col_hbm, z_hbm, out_hbm, hist, col_v):
        cid = lax.axis_index("c")
        sid = lax.axis_index("s")
        wid = sid * NC + cid
        pltpu.sync_copy(z_hbm, hist)
        ones16 = jnp.full((16,), 1.0, dtype=jnp.float32)

        def body(g, carry):
            base = (g * NW + wid) * CH
            pltpu.sync_copy(col_hbm.at[pl.ds(base, CH)], col_v.at[0])
            for j in range(CH // 16):
                idx = col_v[0, pl.ds(j * 16, 16)]
                plsc.addupdate_scatter(hist, [idx], ones16)
            return carry

        lax.fori_loop(0, CPW, body, 0)
        pltpu.sync_copy(hist, out_hbm.at[wid])

    return deg_kernel(col_p, zeros_hn)


def _sc_propagate(hp, row_p, col_p):
    @functools.partial(
        pl.kernel,
        out_type=jax.ShapeDtypeStruct((NC, N, D), jnp.float32),
        mesh=_sc_mesh(),
        scratch_types=[
            pltpu.VMEM_SHARED((ACC_ROWS, D), jnp.float32),
            pltpu.VMEM((1, CH), jnp.int32),
            pltpu.VMEM((1, CH), jnp.int32),
            pltpu.VMEM((CH, D), jnp.float32),
        ],
    )
    def prop_kernel(h_hbm, row_hbm, col_hbm, out_hbm,
                    acc, row_v, col_v, rows_v):
        cid = lax.axis_index("c")
        sid = lax.axis_index("s")
        wid = sid * NC + cid
        base_r = sid * RPT
        pltpu.sync_copy(h_hbm.at[pl.ds(base_r, RPT)],
                        acc.at[pl.ds(base_r, RPT)])

        @pl.when(sid == NS - 1)
        def _():
            pltpu.sync_copy(h_hbm.at[pl.ds(NS * RPT, N - NS * RPT)],
                            acc.at[pl.ds(NS * RPT, N - NS * RPT)])

        plsc.subcore_barrier()

        def body(g, carry):
            base = (g * NW + wid) * CH
            pltpu.sync_copy(row_hbm.at[pl.ds(base, CH)], row_v.at[0])
            pltpu.sync_copy(col_hbm.at[pl.ds(base, CH)], col_v.at[0])
            pltpu.sync_copy(h_hbm.at[row_v.at[0]], rows_v)
            pltpu.sync_copy(rows_v, acc.at[col_v.at[0]], add=True)
            return carry

        lax.fori_loop(0, CPW, body, 0)
        plsc.subcore_barrier()
        pltpu.sync_copy(acc.at[pl.ds(base_r, RPT)],
                        out_hbm.at[cid, pl.ds(base_r, RPT)])

        @pl.when(sid == NS - 1)
        def _():
            pltpu.sync_copy(acc.at[pl.ds(NS * RPT, N - NS * RPT)],
                            out_hbm.at[cid, pl.ds(NS * RPT, N - NS * RPT)])

    return prop_kernel(hp, row_p, col_p)


def _tc_stage1(x, W1, degp):
    def body(x_ref, w_ref, d_ref, h_ref, dis_ref):
        d = jnp.sum(d_ref[...], axis=0)
        dis = lax.rsqrt(d + 1.0)[:, None]
        h = lax.dot_general(x_ref[...], w_ref[...],
                            (((1,), (1,)), ((), ())),
                            preferred_element_type=jnp.float32)
        h_ref[...] = h * dis
        dis_ref[...] = jnp.broadcast_to(dis, (BR, D))

    return pl.pallas_call(
        body,
        grid=(GR,),
        in_specs=[
            pl.BlockSpec((BR, D), lambda i: (i, 0)),
            pl.BlockSpec((D, D), lambda i: (0, 0)),
            pl.BlockSpec((NW, BR), lambda i: (0, i)),
        ],
        out_specs=[
            pl.BlockSpec((BR, D), lambda i: (i, 0)),
            pl.BlockSpec((BR, D), lambda i: (i, 0)),
        ],
        out_shape=[
            jax.ShapeDtypeStruct((N, D), jnp.float32),
            jax.ShapeDtypeStruct((N, D), jnp.float32),
        ],
    )(x, W1, degp)


def _tc_stage2(p, hp, dis, b1, W2):
    def body(p_ref, hp_ref, dis_ref, b_ref, w_ref, o_ref):
        s = p_ref[0] + p_ref[1] - hp_ref[...]
        h = s * dis_ref[...] + b_ref[...]
        h = jnp.maximum(h, 0.0)
        o = lax.dot_general(h, w_ref[...],
                            (((1,), (1,)), ((), ())),
                            preferred_element_type=jnp.float32)
        o_ref[...] = o * dis_ref[...]

    return pl.pallas_call(
        body,
        grid=(GR,),
        in_specs=[
            pl.BlockSpec((NC, BR, D), lambda i: (0, i, 0)),
            pl.BlockSpec((BR, D), lambda i: (i, 0)),
            pl.BlockSpec((BR, D), lambda i: (i, 0)),
            pl.BlockSpec((1, D), lambda i: (0, 0)),
            pl.BlockSpec((D, D), lambda i: (0, 0)),
        ],
        out_specs=pl.BlockSpec((BR, D), lambda i: (i, 0)),
        out_shape=jax.ShapeDtypeStruct((N, D), jnp.float32),
    )(p, hp, dis, b1, W2)


def _tc_stage3(q, hp, dis, b2):
    def body(q_ref, hp_ref, dis_ref, b_ref, o_ref):
        s = q_ref[0] + q_ref[1] - hp_ref[...]
        o_ref[...] = s * dis_ref[...] + b_ref[...]

    return pl.pallas_call(
        body,
        grid=(GR,),
        in_specs=[
            pl.BlockSpec((NC, BR, D), lambda i: (0, i, 0)),
            pl.BlockSpec((BR, D), lambda i: (i, 0)),
            pl.BlockSpec((BR, D), lambda i: (i, 0)),
            pl.BlockSpec((1, D), lambda i: (0, 0)),
        ],
        out_specs=pl.BlockSpec((BR, D), lambda i: (i, 0)),
        out_shape=jax.ShapeDtypeStruct((N, D), jnp.float32),
    )(q, hp, dis, b2)


def kernel(x, edge_index, W1, b1, W2, b2):
    row = edge_index[0].astype(jnp.int32)
    col = edge_index[1].astype(jnp.int32)
    pad = EP2 - E
    row_p = jnp.concatenate([row, jnp.zeros((pad,), jnp.int32)])
    col_p = jnp.concatenate([col, jnp.full((pad,), N, jnp.int32)])

    degp = _sc_degree(col_p, jnp.zeros((HN,), jnp.float32))
    h1p, dis = _tc_stage1(x, W1, degp)
    p = _sc_propagate(h1p, row_p, col_p)
    h2p = _tc_stage2(p, h1p, dis, b1.reshape(1, D), W2)
    q = _sc_propagate(h2p, row_p, col_p)
    return _tc_stage3(q, h2p, dis, b2.reshape(1, D))


# speedup vs baseline: 1.6757x; 1.0558x over previous
import functools

import jax
import jax.numpy as jnp
from jax import lax
from jax.experimental import pallas as pl
from jax.experimental.pallas import tpu as pltpu
from jax.experimental.pallas import tpu_sc as plsc

N = 10000
E = 320000
D = 128
NC = 2
NS = 16
NW = NC * NS
CH = 128
EP = 327680
CPW = EP // CH // NW
EP2 = EP + NW * CH
ACC_ROWS = 10112
RPT = 624
RPT_ACC = ACC_ROWS // NS
BR = 1024
GR = (N + BR - 1) // BR


def _sc_mesh():
    return plsc.VectorSubcoreMesh(core_axis_name="c", subcore_axis_name="s")


HN = 10240


def _sc_degree(col_p, zeros_hn):
    @functools.partial(
        pl.kernel,
        out_type=jax.ShapeDtypeStruct((NW, HN), jnp.float32),
        mesh=_sc_mesh(),
        scratch_types=[
            pltpu.VMEM((HN,), jnp.float32),
            pltpu.VMEM((1, CH), jnp.int32),
        ],
        compiler_params=pltpu.CompilerParams(needs_layout_passes=False),
    )
    def deg_kernel(col_hbm, z_hbm, out_hbm, hist, col_v):
        cid = lax.axis_index("c")
        sid = lax.axis_index("s")
        wid = sid * NC + cid
        pltpu.sync_copy(z_hbm, hist)
        ones16 = jnp.full((16,), 1.0, dtype=jnp.float32)

        def body(g, carry):
            base = (g * NW + wid) * CH
            pltpu.sync_copy(col_hbm.at[pl.ds(base, CH)], col_v.at[0])
            for j in range(CH // 16):
                idx = col_v[0, pl.ds(j * 16, 16)]
                plsc.addupdate_scatter(hist, [idx], ones16)
            return carry

        lax.fori_loop(0, CPW, body, 0)
        pltpu.sync_copy(hist, out_hbm.at[wid])

    return deg_kernel(col_p, zeros_hn)


def _sc_propagate(hp, rc_p):
    @functools.partial(
        pl.kernel,
        out_type=jax.ShapeDtypeStruct((NC, N, D), jnp.float32),
        mesh=_sc_mesh(),
        scratch_types=[
            pltpu.VMEM_SHARED((ACC_ROWS, D), jnp.float32),
            pltpu.VMEM((2, CH), jnp.int32),
            pltpu.VMEM((CH, D), jnp.float32),
        ],
    )
    def prop_kernel(h_hbm, rc_hbm, out_hbm,
                    acc, rc_v, rows_v):
        cid = lax.axis_index("c")
        sid = lax.axis_index("s")
        wid = sid * NC + cid
        base_r = sid * RPT
        pltpu.sync_copy(h_hbm.at[pl.ds(base_r, RPT)],
                        acc.at[pl.ds(base_r, RPT)])

        @pl.when(sid == NS - 1)
        def _():
            pltpu.sync_copy(h_hbm.at[pl.ds(NS * RPT, N - NS * RPT)],
                            acc.at[pl.ds(NS * RPT, N - NS * RPT)])

        plsc.subcore_barrier()

        def body(g, carry):
            pltpu.sync_copy(rc_hbm.at[g * NW + wid], rc_v)
            pltpu.sync_copy(h_hbm.at[rc_v.at[0]], rows_v)
            pltpu.sync_copy(rows_v, acc.at[rc_v.at[1]], add=True)
            return carry

        lax.fori_loop(0, CPW, body, 0)
        plsc.subcore_barrier()
        pltpu.sync_copy(acc.at[pl.ds(base_r, RPT)],
                        out_hbm.at[cid, pl.ds(base_r, RPT)])

        @pl.when(sid == NS - 1)
        def _():
            pltpu.sync_copy(acc.at[pl.ds(NS * RPT, N - NS * RPT)],
                            out_hbm.at[cid, pl.ds(NS * RPT, N - NS * RPT)])

    return prop_kernel(hp, rc_p)


def _tc_stage1(x, W1, degp):
    def body(x_ref, w_ref, d_ref, h_ref, dis_ref):
        d = jnp.sum(d_ref[...], axis=0)
        dis = lax.rsqrt(d + 1.0)[:, None]
        h = lax.dot_general(x_ref[...], w_ref[...],
                            (((1,), (1,)), ((), ())),
                            preferred_element_type=jnp.float32)
        h_ref[...] = h * dis
        dis_ref[...] = jnp.broadcast_to(dis, (BR, D))

    return pl.pallas_call(
        body,
        grid=(GR,),
        in_specs=[
            pl.BlockSpec((BR, D), lambda i: (i, 0)),
            pl.BlockSpec((D, D), lambda i: (0, 0)),
            pl.BlockSpec((NW, BR), lambda i: (0, i)),
        ],
        out_specs=[
            pl.BlockSpec((BR, D), lambda i: (i, 0)),
            pl.BlockSpec((BR, D), lambda i: (i, 0)),
        ],
        out_shape=[
            jax.ShapeDtypeStruct((N, D), jnp.float32),
            jax.ShapeDtypeStruct((N, D), jnp.float32),
        ],
    )(x, W1, degp)


def _tc_stage2(p, hp, dis, b1, W2):
    def body(p_ref, hp_ref, dis_ref, b_ref, w_ref, o_ref):
        s = p_ref[0] + p_ref[1] - hp_ref[...]
        h = s * dis_ref[...] + b_ref[...]
        h = jnp.maximum(h, 0.0)
        o = lax.dot_general(h, w_ref[...],
                            (((1,), (1,)), ((), ())),
                            preferred_element_type=jnp.float32)
        o_ref[...] = o * dis_ref[...]

    return pl.pallas_call(
        body,
        grid=(GR,),
        in_specs=[
            pl.BlockSpec((NC, BR, D), lambda i: (0, i, 0)),
            pl.BlockSpec((BR, D), lambda i: (i, 0)),
            pl.BlockSpec((BR, D), lambda i: (i, 0)),
            pl.BlockSpec((1, D), lambda i: (0, 0)),
            pl.BlockSpec((D, D), lambda i: (0, 0)),
        ],
        out_specs=pl.BlockSpec((BR, D), lambda i: (i, 0)),
        out_shape=jax.ShapeDtypeStruct((N, D), jnp.float32),
    )(p, hp, dis, b1, W2)


def _tc_stage3(q, hp, dis, b2):
    def body(q_ref, hp_ref, dis_ref, b_ref, o_ref):
        s = q_ref[0] + q_ref[1] - hp_ref[...]
        o_ref[...] = s * dis_ref[...] + b_ref[...]

    return pl.pallas_call(
        body,
        grid=(GR,),
        in_specs=[
            pl.BlockSpec((NC, BR, D), lambda i: (0, i, 0)),
            pl.BlockSpec((BR, D), lambda i: (i, 0)),
            pl.BlockSpec((BR, D), lambda i: (i, 0)),
            pl.BlockSpec((1, D), lambda i: (0, 0)),
        ],
        out_specs=pl.BlockSpec((BR, D), lambda i: (i, 0)),
        out_shape=jax.ShapeDtypeStruct((N, D), jnp.float32),
    )(q, hp, dis, b2)


def kernel(x, edge_index, W1, b1, W2, b2):
    row = edge_index[0].astype(jnp.int32)
    col = edge_index[1].astype(jnp.int32)
    pad = EP2 - E
    row_p = jnp.concatenate([row, jnp.zeros((pad,), jnp.int32)])
    col_p = jnp.concatenate([col, jnp.full((pad,), N, jnp.int32)])
    rc_p = jnp.stack([row_p.reshape(EP2 // CH, CH),
                      col_p.reshape(EP2 // CH, CH)], axis=1)

    degp = _sc_degree(col_p, jnp.zeros((HN,), jnp.float32))
    h1p, dis = _tc_stage1(x, W1, degp)
    p = _sc_propagate(h1p, rc_p)
    h2p = _tc_stage2(p, h1p, dis, b1.reshape(1, D), W2)
    q = _sc_propagate(h2p, rc_p)
    return _tc_stage3(q, h2p, dis, b2.reshape(1, D))
